# baseline (speedup 1.0000x reference)
import functools

import jax
import jax.numpy as jnp
from jax import lax
from jax.experimental import pallas as pl
from jax.experimental.pallas import tpu as pltpu
from jax.experimental.pallas import tpu_sc as plsc

VOCAB = 100000
D = 128
T = 192
B = 1024
N = B * T
NC = 2
NS = 16
NW = NC * NS
ROWS_PER_W = N // NW
CHUNK = 64
NCH = ROWS_PER_W // CHUNK
NB = 4
LANES = 16
VPR = D // LANES


def _body(ids_hbm, table_hbm, tpe_hbm, role_hbm,
          latent_hbm, raw_hbm,
          idx_v, rows0, rows1, rows2, rows3, lat0, lat1, lat2, lat3,
          tpe_v, role_v, pos_v,
          gsem0, gsem1, gsem2, gsem3, ssem0, ssem1, ssem2, ssem3):
    wid = lax.axis_index("s") * NC + lax.axis_index("c")
    w_base = wid * ROWS_PER_W
    rows_v = (rows0, rows1, rows2, rows3)
    lat_v = (lat0, lat1, lat2, lat3)
    gsem = (gsem0, gsem1, gsem2, gsem3)
    ssem = (ssem0, ssem1, ssem2, ssem3)

    pltpu.sync_copy(ids_hbm.at[pl.ds(wid * NCH, NCH)], idx_v)

    pltpu.async_copy(table_hbm.at[idx_v.at[0]], rows_v[0], gsem[0])
    pltpu.async_copy(table_hbm.at[idx_v.at[1]], rows_v[1], gsem[1])

    pltpu.sync_copy(tpe_hbm, tpe_v)
    pltpu.sync_copy(role_hbm, role_v)

    @plsc.parallel_loop(0, T // 3, step=1, unroll=2)
    def build_pos(k):
        for r in range(3):
            for j in range(VPR):
                sl = pl.ds(j * LANES, LANES)
                pos_v[k * 3 + r, sl] = tpe_v[k, sl] + role_v[r, sl]

    def gather_wait(b):
        pltpu.make_async_copy(table_hbm.at[idx_v.at[0]], rows_v[b], gsem[b]).wait()

    def scatter_wait(b):
        pltpu.make_async_copy(rows_v[b], raw_hbm.at[pl.ds(0, CHUNK)], ssem[b]).wait()
        pltpu.make_async_copy(lat_v[b], latent_hbm.at[pl.ds(0, CHUNK)], ssem[b]).wait()

    def outer(i, _):
        for b in range(NB):
            g = NB * i + b

            gather_wait(b)

            base = w_base + g * CHUNK
            pltpu.async_copy(rows_v[b], raw_hbm.at[pl.ds(base, CHUNK)], ssem[b])

            pos_off = (g % 3) * CHUNK

            @plsc.parallel_loop(0, CHUNK, step=1, unroll=4)
            def add_row(r):
                for j in range(VPR):
                    sl = pl.ds(j * LANES, LANES)
                    lat_v[b][r, sl] = rows_v[b][r, sl] + pos_v[pos_off + r, sl]

            pltpu.async_copy(lat_v[b], latent_hbm.at[pl.ds(base, CHUNK)], ssem[b])

            b2 = (b + 2) % NB

            @pl.when(g >= 2)
            def _():
                scatter_wait(b2)

            @pl.when(g + 2 < NCH)
            def _():
                pltpu.async_copy(table_hbm.at[idx_v.at[g + 2]], rows_v[b2], gsem[b2])
        return 0

    lax.fori_loop(0, NCH // NB, outer, 0, unroll=False)

    scatter_wait((NCH - 2) % NB)
    scatter_wait((NCH - 1) % NB)


@jax.jit
def _run(ids_2d, token_emb, triple_pos_emb, role_emb):
    mesh = plsc.VectorSubcoreMesh(core_axis_name="c", subcore_axis_name="s")
    f = pl.kernel(
        _body,
        out_type=(
            jax.ShapeDtypeStruct((N, D), jnp.float32),
            jax.ShapeDtypeStruct((N, D), jnp.float32),
        ),
        mesh=mesh,
        scratch_types=[
            pltpu.VMEM((NCH, CHUNK), jnp.int32),
            pltpu.VMEM((CHUNK, D), jnp.float32),
            pltpu.VMEM((CHUNK, D), jnp.float32),
            pltpu.VMEM((CHUNK, D), jnp.float32),
            pltpu.VMEM((CHUNK, D), jnp.float32),
            pltpu.VMEM((CHUNK, D), jnp.float32),
            pltpu.VMEM((CHUNK, D), jnp.float32),
            pltpu.VMEM((CHUNK, D), jnp.float32),
            pltpu.VMEM((CHUNK, D), jnp.float32),
            pltpu.VMEM((T // 3, D), jnp.float32),
            pltpu.VMEM((3, D), jnp.float32),
            pltpu.VMEM((T, D), jnp.float32),
            pltpu.SemaphoreType.DMA,
            pltpu.SemaphoreType.DMA,
            pltpu.SemaphoreType.DMA,
            pltpu.SemaphoreType.DMA,
            pltpu.SemaphoreType.DMA,
            pltpu.SemaphoreType.DMA,
            pltpu.SemaphoreType.DMA,
            pltpu.SemaphoreType.DMA,
        ],
    )
    return f(ids_2d, token_emb, triple_pos_emb, role_emb)


def kernel(input_ids, token_emb, triple_pos_emb, role_emb):
    ids_2d = input_ids.reshape(N // CHUNK, CHUNK).astype(jnp.int32)
    latent, raw = _run(ids_2d, token_emb, triple_pos_emb, role_emb)
    return latent.reshape(B, T, D), raw.reshape(B, T, D)


# speedup vs baseline: 1.0132x; 1.0132x over previous
import functools

import jax
import jax.numpy as jnp
from jax import lax
from jax.experimental import pallas as pl
from jax.experimental.pallas import tpu as pltpu
from jax.experimental.pallas import tpu_sc as plsc

VOCAB = 100000
D = 128
T = 192
B = 1024
N = B * T
NC = 2
NS = 16
NW = NC * NS
ROWS_PER_W = N // NW
CHUNK = 96
NCH = ROWS_PER_W // CHUNK
NB = 3
LANES = 16
VPR = D // LANES


def _body(ids_hbm, table_hbm, tpe_hbm, role_hbm,
          latent_hbm, raw_hbm,
          idx_v, rows0, rows1, rows2, lat0, lat1, lat2, tpe_v, role_v, pos_v,
          gsem0, gsem1, gsem2, ssem0, ssem1, ssem2):
    wid = lax.axis_index("s") * NC + lax.axis_index("c")
    w_base = wid * ROWS_PER_W
    rows_v = (rows0, rows1, rows2)
    lat_v = (lat0, lat1, lat2)
    gsem = (gsem0, gsem1, gsem2)
    ssem = (ssem0, ssem1, ssem2)

    pltpu.sync_copy(ids_hbm.at[pl.ds(wid * NCH, NCH)], idx_v)

    pltpu.async_copy(table_hbm.at[idx_v.at[0]], rows_v[0], gsem[0])
    pltpu.async_copy(table_hbm.at[idx_v.at[1]], rows_v[1], gsem[1])

    pltpu.sync_copy(tpe_hbm, tpe_v)
    pltpu.sync_copy(role_hbm, role_v)

    @plsc.parallel_loop(0, T // 3, step=1, unroll=2)
    def build_pos(k):
        for r in range(3):
            for j in range(VPR):
                sl = pl.ds(j * LANES, LANES)
                pos_v[k * 3 + r, sl] = tpe_v[k, sl] + role_v[r, sl]

    def gather_wait(b):
        pltpu.make_async_copy(table_hbm.at[idx_v.at[0]], rows_v[b], gsem[b]).wait()

    def scatter_wait(b):
        pltpu.make_async_copy(rows_v[b], raw_hbm.at[pl.ds(0, CHUNK)], ssem[b]).wait()
        pltpu.make_async_copy(lat_v[b], latent_hbm.at[pl.ds(0, CHUNK)], ssem[b]).wait()

    def outer(i, _):
        for b in range(NB):
            g = NB * i + b

            gather_wait(b)

            base = w_base + g * CHUNK
            pltpu.async_copy(rows_v[b], raw_hbm.at[pl.ds(base, CHUNK)], ssem[b])

            pos_off = (g % 2) * CHUNK

            @plsc.parallel_loop(0, CHUNK, step=1, unroll=2)
            def add_row(r):
                for j in range(VPR):
                    sl = pl.ds(j * LANES, LANES)
                    lat_v[b][r, sl] = rows_v[b][r, sl] + pos_v[pos_off + r, sl]

            pltpu.async_copy(lat_v[b], latent_hbm.at[pl.ds(base, CHUNK)], ssem[b])

            b2 = (b + 2) % NB

            @pl.when(g >= 1)
            def _():
                scatter_wait(b2)

            @pl.when(g + 2 < NCH)
            def _():
                pltpu.async_copy(table_hbm.at[idx_v.at[g + 2]], rows_v[b2], gsem[b2])
        return 0

    lax.fori_loop(0, NCH // NB, outer, 0, unroll=False)

    g_last = NCH - 1
    b = g_last % NB
    gather_wait(b)
    base = w_base + g_last * CHUNK
    pltpu.async_copy(rows_v[b], raw_hbm.at[pl.ds(base, CHUNK)], ssem[b])

    @plsc.parallel_loop(0, CHUNK, step=1, unroll=2)
    def add_last(r):
        for j in range(VPR):
            sl = pl.ds(j * LANES, LANES)
            lat_v[b][r, sl] = rows_v[b][r, sl] + pos_v[CHUNK + r, sl]

    pltpu.async_copy(lat_v[b], latent_hbm.at[pl.ds(base, CHUNK)], ssem[b])
    scatter_wait((g_last + 2) % NB)
    scatter_wait(b)


@jax.jit
def _run(ids_2d, token_emb, triple_pos_emb, role_emb):
    mesh = plsc.VectorSubcoreMesh(core_axis_name="c", subcore_axis_name="s")
    f = pl.kernel(
        _body,
        out_type=(
            jax.ShapeDtypeStruct((N, D), jnp.float32),
            jax.ShapeDtypeStruct((N, D), jnp.float32),
        ),
        mesh=mesh,
        scratch_types=[
            pltpu.VMEM((NCH, CHUNK), jnp.int32),
            pltpu.VMEM((CHUNK, D), jnp.float32),
            pltpu.VMEM((CHUNK, D), jnp.float32),
            pltpu.VMEM((CHUNK, D), jnp.float32),
            pltpu.VMEM((CHUNK, D), jnp.float32),
            pltpu.VMEM((CHUNK, D), jnp.float32),
            pltpu.VMEM((CHUNK, D), jnp.float32),
            pltpu.VMEM((T // 3, D), jnp.float32),
            pltpu.VMEM((3, D), jnp.float32),
            pltpu.VMEM((T, D), jnp.float32),
            pltpu.SemaphoreType.DMA,
            pltpu.SemaphoreType.DMA,
            pltpu.SemaphoreType.DMA,
            pltpu.SemaphoreType.DMA,
            pltpu.SemaphoreType.DMA,
            pltpu.SemaphoreType.DMA,
        ],
    )
    return f(ids_2d, token_emb, triple_pos_emb, role_emb)


def kernel(input_ids, token_emb, triple_pos_emb, role_emb):
    ids_2d = input_ids.reshape(N // CHUNK, CHUNK).astype(jnp.int32)
    latent, raw = _run(ids_2d, token_emb, triple_pos_emb, role_emb)
    return latent.reshape(B, T, D), raw.reshape(B, T, D)


# speedup vs baseline: 1.0139x; 1.0007x over previous
import functools

import jax
import jax.numpy as jnp
from jax import lax
from jax.experimental import pallas as pl
from jax.experimental.pallas import tpu as pltpu
from jax.experimental.pallas import tpu_sc as plsc

VOCAB = 100000
D = 128
T = 192
B = 1024
N = B * T
NC = 2
NS = 16
NW = NC * NS
ROWS_PER_W = N // NW
CHUNK = 128
NCH = ROWS_PER_W // CHUNK
NR = 3
NL = 2
POS_ROWS = T + CHUNK // 2
POS_OFF = (0, 128, 64)
LANES = 16
VPR = D // LANES


def _body(ids_hbm, table_hbm, tpe_hbm, role_hbm,
          latent_hbm, raw_hbm,
          idx_v, rows0, rows1, rows2, lat0, lat1, tpe_v, role_v, pos_v,
          gsem0, gsem1, gsem2, rsem0, rsem1, rsem2, lsem0, lsem1):
    wid = lax.axis_index("s") * NC + lax.axis_index("c")
    w_base = wid * ROWS_PER_W
    rows_v = (rows0, rows1, rows2)
    lat_v = (lat0, lat1)
    gsem = (gsem0, gsem1, gsem2)
    rsem = (rsem0, rsem1, rsem2)
    lsem = (lsem0, lsem1)

    pltpu.sync_copy(ids_hbm.at[pl.ds(wid * NCH, NCH)], idx_v)

    pltpu.async_copy(table_hbm.at[idx_v.at[0]], rows_v[0], gsem[0])
    pltpu.async_copy(table_hbm.at[idx_v.at[1]], rows_v[1], gsem[1])

    pltpu.sync_copy(tpe_hbm, tpe_v)
    pltpu.sync_copy(role_hbm, role_v)

    @plsc.parallel_loop(0, T // 3, step=1, unroll=2)
    def build_pos(k):
        for r in range(3):
            for j in range(VPR):
                sl = pl.ds(j * LANES, LANES)
                pos_v[k * 3 + r, sl] = tpe_v[k, sl] + role_v[r, sl]

    @plsc.parallel_loop(0, POS_ROWS - T, step=1, unroll=2)
    def wrap_pos(t):
        for j in range(VPR):
            sl = pl.ds(j * LANES, LANES)
            pos_v[T + t, sl] = pos_v[t, sl]

    def gather_wait(br):
        pltpu.make_async_copy(table_hbm.at[idx_v.at[0]], rows_v[br], gsem[br]).wait()

    def raw_wait(br):
        pltpu.make_async_copy(rows_v[br], raw_hbm.at[pl.ds(0, CHUNK)], rsem[br]).wait()

    def lat_wait(bl):
        pltpu.make_async_copy(lat_v[bl], latent_hbm.at[pl.ds(0, CHUNK)], lsem[bl]).wait()

    def outer(i, _):
        for b in range(NR * NL):
            g = NR * NL * i + b
            br = b % NR
            bl = b % NL

            gather_wait(br)

            base = w_base + g * CHUNK
            pltpu.async_copy(rows_v[br], raw_hbm.at[pl.ds(base, CHUNK)], rsem[br])

            @pl.when(g >= 2)
            def _():
                lat_wait(bl)

            pos_off = POS_OFF[b % 3]

            @plsc.parallel_loop(0, CHUNK, step=1, unroll=4)
            def add_row(r):
                for j in range(VPR):
                    sl = pl.ds(j * LANES, LANES)
                    lat_v[bl][r, sl] = rows_v[br][r, sl] + pos_v[pos_off + r, sl]

            pltpu.async_copy(lat_v[bl], latent_hbm.at[pl.ds(base, CHUNK)], lsem[bl])

            br2 = (br + 2) % NR

            @pl.when(g >= 1)
            def _():
                raw_wait(br2)

            @pl.when(g + 2 < NCH)
            def _():
                pltpu.async_copy(table_hbm.at[idx_v.at[g + 2]], rows_v[br2], gsem[br2])
        return 0

    lax.fori_loop(0, NCH // (NR * NL), outer, 0, unroll=False)

    raw_wait((NCH - 1) % NR)
    lat_wait((NCH - 2) % NL)
    lat_wait((NCH - 1) % NL)


@jax.jit
def _run(ids_2d, token_emb, triple_pos_emb, role_emb):
    mesh = plsc.VectorSubcoreMesh(core_axis_name="c", subcore_axis_name="s")
    f = pl.kernel(
        _body,
        out_type=(
            jax.ShapeDtypeStruct((N, D), jnp.float32),
            jax.ShapeDtypeStruct((N, D), jnp.float32),
        ),
        mesh=mesh,
        scratch_types=[
            pltpu.VMEM((NCH, CHUNK), jnp.int32),
            pltpu.VMEM((CHUNK, D), jnp.float32),
            pltpu.VMEM((CHUNK, D), jnp.float32),
            pltpu.VMEM((CHUNK, D), jnp.float32),
            pltpu.VMEM((CHUNK, D), jnp.float32),
            pltpu.VMEM((CHUNK, D), jnp.float32),
            pltpu.VMEM((T // 3, D), jnp.float32),
            pltpu.VMEM((3, D), jnp.float32),
            pltpu.VMEM((POS_ROWS, D), jnp.float32),
            pltpu.SemaphoreType.DMA,
            pltpu.SemaphoreType.DMA,
            pltpu.SemaphoreType.DMA,
            pltpu.SemaphoreType.DMA,
            pltpu.SemaphoreType.DMA,
            pltpu.SemaphoreType.DMA,
            pltpu.SemaphoreType.DMA,
            pltpu.SemaphoreType.DMA,
        ],
    )
    return f(ids_2d, token_emb, triple_pos_emb, role_emb)


def kernel(input_ids, token_emb, triple_pos_emb, role_emb):
    ids_2d = input_ids.reshape(N // CHUNK, CHUNK).astype(jnp.int32)
    latent, raw = _run(ids_2d, token_emb, triple_pos_emb, role_emb)
    return latent.reshape(B, T, D), raw.reshape(B, T, D)
